# baseline (speedup 1.0000x reference)
import jax
import jax.numpy as jnp
from jax.experimental import pallas as pl
from jax.experimental.pallas import tpu as pltpu

_SCORE_WEIGHT = 0.5
_MAX_ITER = 20
_PROJ_ITER = 5
_RELAX_LR = 0.1
_EPS = 1e-8

_NC = 18
_VMEM_LIMIT = 48 * 1024 * 1024


def _pass1_body(a_ref, b_ref, inter_ref, asum_ref, bsum_ref):
    j = pl.program_id(1)
    o, ch = a_ref.shape
    p = b_ref.shape[0]

    @pl.when(j == 0)
    def _():
        inter_ref[...] = jnp.zeros_like(inter_ref)
        asum_ref[...] = jnp.zeros_like(asum_ref)
        bsum_ref[...] = jnp.zeros_like(bsum_ref)

    a = a_ref[...]
    lhs = jnp.concatenate(
        [a.astype(jnp.bfloat16), jnp.ones((8, ch), jnp.bfloat16)], axis=0)
    bb = b_ref[...].astype(jnp.bfloat16)
    acc = jax.lax.dot_general(lhs, bb, (((1,), (1,)), ((), ())),
                              preferred_element_type=jnp.float32)
    inter_ref[...] += acc[None, :o, :]
    bsum_ref[...] += acc[None, o:o + 1, :]
    asum_ref[...] += jnp.sum(a, axis=1, keepdims=True)[None]


def _pass2_body(inter_ref, asum_ref, bsum_ref, pf_ref, tf_ref, ps_ref,
                binx_ref, ms_ref, ds_ref):
    o = inter_ref.shape[1]
    p = inter_ref.shape[2]
    inter = inter_ref[0] + inter_ref[1]
    asum = asum_ref[0] + asum_ref[1]
    bsum = bsum_ref[0] + bsum_ref[1]
    union = asum + bsum - inter
    iou = inter / (union + _EPS)

    pf = pf_ref[...]
    kf = pf / (jnp.sqrt(jnp.sum(pf * pf, axis=1, keepdims=True)) + _EPS)
    tf = tf_ref[...]
    qn = jnp.sqrt(jnp.sum(tf * tf, axis=2, keepdims=True)) + _EPS
    qf = tf / qn
    qsum = jnp.sum(qf, axis=0)
    feature_sim = jax.lax.dot_general(
        qsum, kf, (((1,), (1,)), ((), ())),
        preferred_element_type=jnp.float32) / tf_ref.shape[0]

    sim = feature_sim * (1.0 - _SCORE_WEIGHT) + iou * _SCORE_WEIGHT
    cost = -sim

    x0 = jnp.full((o, p), 1.0 / p, dtype=jnp.float32)

    def proj_body(_, x):
        x = jnp.clip(x, 0.0, 1.0)
        return x / (jnp.sum(x, axis=1, keepdims=True) + _EPS)

    def outer(_, carry):
        x, s = carry
        xn = jax.lax.fori_loop(0, _PROJ_ITER, proj_body, x - _RELAX_LR * cost)
        return xn, s + xn

    _, s = jax.lax.fori_loop(
        0, _MAX_ITER, outer, (x0, jnp.zeros((o, p), dtype=jnp.float32)))
    ridx = s / jnp.float32(_MAX_ITER)

    logic = (ridx > 0.01).astype(jnp.float32)
    binx = ridx * logic
    binx_ref[...] = binx
    ms_ref[...] = jnp.max(jnp.clip(ridx, 0.0, 1.0) * sim, axis=1,
                          keepdims=True)
    ds_ref[...] = jnp.sum(ps_ref[...] * binx, axis=1, keepdims=True)


def _pass3_body(binx_ref, b_ref, out_ref):
    x = binx_ref[...]
    xh = x.astype(jnp.bfloat16)
    xl = (x - xh.astype(jnp.float32)).astype(jnp.bfloat16)
    bb = b_ref[...].astype(jnp.bfloat16)
    dn = (((1,), (0,)), ((), ()))
    out_ref[...] = (
        jax.lax.dot_general(xh, bb, dn, preferred_element_type=jnp.float32)
        + jax.lax.dot_general(xl, bb, dn, preferred_element_type=jnp.float32))


def kernel(proposed_feature, proposed_mask, template_feature,
           mask_last_occurence, proposal_score):
    p, d = proposed_feature.shape
    o = mask_last_occurence.shape[0]
    h, w = proposed_mask.shape[1], proposed_mask.shape[2]
    hw = h * w
    ch = hw // _NC
    half = _NC // 2

    b2 = jax.lax.optimization_barrier(proposed_mask.reshape(p, hw))
    a2 = jax.lax.optimization_barrier(mask_last_occurence.reshape(o, hw))

    inter_p, asum_p, bsum_p = pl.pallas_call(
        _pass1_body,
        grid=(2, half),
        in_specs=[
            pl.BlockSpec((o, ch), lambda i, j: (0, i * half + j)),
            pl.BlockSpec((p, ch), lambda i, j: (0, i * half + j)),
        ],
        out_specs=[
            pl.BlockSpec((1, o, p), lambda i, j: (i, 0, 0)),
            pl.BlockSpec((1, o, 1), lambda i, j: (i, 0, 0)),
            pl.BlockSpec((1, 1, p), lambda i, j: (i, 0, 0)),
        ],
        out_shape=[
            jax.ShapeDtypeStruct((2, o, p), jnp.float32),
            jax.ShapeDtypeStruct((2, o, 1), jnp.float32),
            jax.ShapeDtypeStruct((2, 1, p), jnp.float32),
        ],
        compiler_params=pltpu.CompilerParams(
            dimension_semantics=("parallel", "arbitrary"),
            vmem_limit_bytes=_VMEM_LIMIT),
        name="mask_inter",
    )(a2, b2)

    binx, ms, ds = pl.pallas_call(
        _pass2_body,
        out_shape=[
            jax.ShapeDtypeStruct((o, p), jnp.float32),
            jax.ShapeDtypeStruct((o, 1), jnp.float32),
            jax.ShapeDtypeStruct((o, 1), jnp.float32),
        ],
        name="match_solve",
    )(inter_p, asum_p, bsum_p, proposed_feature, template_feature,
      proposal_score.reshape(1, p))

    outmask = pl.pallas_call(
        _pass3_body,
        grid=(2, half),
        in_specs=[
            pl.BlockSpec((o, p), lambda i, j: (0, 0)),
            pl.BlockSpec((p, ch), lambda i, j: (0, i * half + j)),
        ],
        out_specs=pl.BlockSpec((o, ch), lambda i, j: (0, i * half + j)),
        out_shape=jax.ShapeDtypeStruct((o, hw), jnp.float32),
        compiler_params=pltpu.CompilerParams(
            dimension_semantics=("parallel", "arbitrary"),
            vmem_limit_bytes=_VMEM_LIMIT),
        name="outmask",
    )(binx, b2)

    return (outmask.reshape(o, h, w), ms.reshape(o), ds.reshape(o))


# speedup vs baseline: 1.3644x; 1.3644x over previous
import jax
import jax.numpy as jnp
from jax.experimental import pallas as pl
from jax.experimental.pallas import tpu as pltpu

_SCORE_WEIGHT = 0.5
_MAX_ITER = 20
_PROJ_ITER = 5
_RELAX_LR = 0.1
_EPS = 1e-8

_HB = 16
_VMEM_LIMIT = 48 * 1024 * 1024


def _pass1_body(a_ref, b_ref, inter_ref, asum_ref, bsum_ref):
    j = pl.program_id(0)
    o, hb, w = a_ref.shape
    p, ch = b_ref.shape

    @pl.when(j == 0)
    def _():
        inter_ref[...] = jnp.zeros_like(inter_ref)
        asum_ref[...] = jnp.zeros_like(asum_ref)
        bsum_ref[...] = jnp.zeros_like(bsum_ref)

    a = a_ref[...].reshape(o, hb * w)
    lhs = jnp.concatenate(
        [a.astype(jnp.bfloat16), jnp.ones((8, ch), jnp.bfloat16)], axis=0)
    bb = b_ref[...].astype(jnp.bfloat16)
    acc = jax.lax.dot_general(lhs, bb, (((1,), (1,)), ((), ())),
                              preferred_element_type=jnp.float32)
    inter_ref[...] += acc[None, :o, :]
    bsum_ref[...] += acc[None, o:o + 1, :]
    asum_ref[...] += jnp.sum(a, axis=1, keepdims=True)[None]


def _pass2_body(inter_ref, asum_ref, bsum_ref, pf_ref, tf_ref, ps_ref,
                binx_ref, ms_ref, ds_ref):
    p = inter_ref.shape[2]
    inter = inter_ref[0]
    asum = asum_ref[0]
    bsum = bsum_ref[0]
    union = asum + bsum - inter
    iou = inter / (union + _EPS)

    pf = pf_ref[...]
    kf = pf / (jnp.sqrt(jnp.sum(pf * pf, axis=1, keepdims=True)) + _EPS)
    tf = tf_ref[...]
    qn = jnp.sqrt(jnp.sum(tf * tf, axis=2, keepdims=True)) + _EPS
    qf = tf / qn
    qsum = jnp.sum(qf, axis=0)
    feature_sim = jax.lax.dot_general(
        qsum, kf, (((1,), (1,)), ((), ())),
        preferred_element_type=jnp.float32) / tf_ref.shape[0]

    sim = feature_sim * (1.0 - _SCORE_WEIGHT) + iou * _SCORE_WEIGHT
    cost = -sim

    ob = inter_ref.shape[1]
    x0 = jnp.full((ob, p), 1.0 / p, dtype=jnp.float32)

    def proj_body(_, x):
        x = jnp.clip(x, 0.0, 1.0)
        return x / (jnp.sum(x, axis=1, keepdims=True) + _EPS)

    def outer(_, carry):
        x, s = carry
        xn = jax.lax.fori_loop(0, _PROJ_ITER, proj_body, x - _RELAX_LR * cost)
        return xn, s + xn

    _, s = jax.lax.fori_loop(
        0, _MAX_ITER, outer, (x0, jnp.zeros((ob, p), dtype=jnp.float32)))
    ridx = s / jnp.float32(_MAX_ITER)

    logic = (ridx > 0.01).astype(jnp.float32)
    binx = ridx * logic
    binx_ref[...] = binx
    ms_ref[...] = jnp.max(jnp.clip(ridx, 0.0, 1.0) * sim, axis=1,
                          keepdims=True)
    ds_ref[...] = jnp.sum(ps_ref[...] * binx, axis=1, keepdims=True)


def _pass3_body(binx_ref, b_ref, out_ref):
    o, hb, w = out_ref.shape
    x = binx_ref[...]
    xh = x.astype(jnp.bfloat16)
    xl = (x - xh.astype(jnp.float32)).astype(jnp.bfloat16)
    bb = b_ref[...].astype(jnp.bfloat16)
    dn = (((1,), (0,)), ((), ()))
    flat = (jax.lax.dot_general(xh, bb, dn, preferred_element_type=jnp.float32)
            + jax.lax.dot_general(xl, bb, dn,
                                  preferred_element_type=jnp.float32))
    out_ref[...] = flat.reshape(o, hb, w)


def kernel(proposed_feature, proposed_mask, template_feature,
           mask_last_occurence, proposal_score):
    p, d = proposed_feature.shape
    o = mask_last_occurence.shape[0]
    h, w = proposed_mask.shape[1], proposed_mask.shape[2]
    hw = h * w
    ch = _HB * w
    nchunks = h // _HB

    b2 = jax.lax.optimization_barrier(proposed_mask.reshape(p, hw))

    inter_p, asum_p, bsum_p = pl.pallas_call(
        _pass1_body,
        grid=(nchunks,),
        in_specs=[
            pl.BlockSpec((o, _HB, w), lambda j: (0, j, 0)),
            pl.BlockSpec((p, ch), lambda j: (0, j)),
        ],
        out_specs=[
            pl.BlockSpec((1, o, p), lambda j: (0, 0, 0)),
            pl.BlockSpec((1, o, 1), lambda j: (0, 0, 0)),
            pl.BlockSpec((1, 1, p), lambda j: (0, 0, 0)),
        ],
        out_shape=[
            jax.ShapeDtypeStruct((1, o, p), jnp.float32),
            jax.ShapeDtypeStruct((1, o, 1), jnp.float32),
            jax.ShapeDtypeStruct((1, 1, p), jnp.float32),
        ],
        compiler_params=pltpu.CompilerParams(
            dimension_semantics=("arbitrary",),
            vmem_limit_bytes=_VMEM_LIMIT),
        name="mask_inter",
    )(mask_last_occurence, b2)

    binx, ms, ds = pl.pallas_call(
        _pass2_body,
        out_shape=[
            jax.ShapeDtypeStruct((o, p), jnp.float32),
            jax.ShapeDtypeStruct((o, 1), jnp.float32),
            jax.ShapeDtypeStruct((o, 1), jnp.float32),
        ],
        name="match_solve",
    )(inter_p, asum_p, bsum_p, proposed_feature, template_feature,
      proposal_score.reshape(1, p))

    outmask = pl.pallas_call(
        _pass3_body,
        grid=(nchunks,),
        in_specs=[
            pl.BlockSpec((o, p), lambda j: (0, 0)),
            pl.BlockSpec((p, ch), lambda j: (0, j)),
        ],
        out_specs=pl.BlockSpec((o, _HB, w), lambda j: (0, j, 0)),
        out_shape=jax.ShapeDtypeStruct((o, h, w), jnp.float32),
        compiler_params=pltpu.CompilerParams(
            dimension_semantics=("arbitrary",),
            vmem_limit_bytes=_VMEM_LIMIT),
        name="outmask",
    )(binx, b2)

    return (outmask, ms.reshape(o), ds.reshape(o))


# speedup vs baseline: 1.5132x; 1.1091x over previous
import jax
import jax.numpy as jnp
from jax.experimental import pallas as pl
from jax.experimental.pallas import tpu as pltpu

_SCORE_WEIGHT = 0.5
_MAX_ITER = 20
_PROJ_ITER = 5
_RELAX_LR = 0.1
_EPS = 1e-8

_HB = 16
_VMEM_LIMIT = 52 * 1024 * 1024


def _fused_body(a_ref, b_ref, pf_ref, tf_ref, ps_ref,
                out_ref, ms_ref, ds_ref,
                b8_scr, inter_scr, asum_scr, binx_scr):
    ph = pl.program_id(0)
    j = pl.program_id(1)
    o, hb, w = a_ref.shape
    p, ch = b_ref.shape
    half = p // 2

    @pl.when((ph == 0) & (j == 0))
    def _():
        inter_scr[...] = jnp.zeros_like(inter_scr)
        asum_scr[...] = jnp.zeros_like(asum_scr)

    @pl.when(ph == 0)
    def _phase0():
        a = a_ref[...].reshape(o, hb * w)
        lhs = jnp.concatenate(
            [a.astype(jnp.bfloat16), jnp.ones((8, ch), jnp.bfloat16)],
            axis=0)
        bb32 = b_ref[...]
        bbf = bb32.astype(jnp.bfloat16)
        acc = jax.lax.dot_general(lhs, bbf, (((1,), (1,)), ((), ())),
                                  preferred_element_type=jnp.float32)
        inter_scr[...] += acc
        asum_scr[...] += jnp.sum(a, axis=1, keepdims=True)
        b8 = bb32.astype(jnp.int8)
        b8_scr[j, :half, :] = b8[:half, :]
        b8_scr[j, half:, :] = b8[half:, :]

    @pl.when((ph == 1) & (j == 0))
    def _solve():
        inter = inter_scr[:o, :]
        bsum = inter_scr[o:o + 1, :]
        asum = asum_scr[...]
        union = asum + bsum - inter
        iou = inter / (union + _EPS)

        pf = pf_ref[...]
        kf = pf / (jnp.sqrt(jnp.sum(pf * pf, axis=1, keepdims=True)) + _EPS)
        tf = tf_ref[...]
        qn = jnp.sqrt(jnp.sum(tf * tf, axis=2, keepdims=True)) + _EPS
        qf = tf / qn
        qsum = jnp.sum(qf, axis=0)
        feature_sim = jax.lax.dot_general(
            qsum, kf, (((1,), (1,)), ((), ())),
            preferred_element_type=jnp.float32) / tf_ref.shape[0]

        sim = feature_sim * (1.0 - _SCORE_WEIGHT) + iou * _SCORE_WEIGHT
        cost = -sim

        x0 = jnp.full((o, p), 1.0 / p, dtype=jnp.float32)

        def proj_body(_, x):
            x = jnp.clip(x, 0.0, 1.0)
            return x / (jnp.sum(x, axis=1, keepdims=True) + _EPS)

        def outer(_, carry):
            x, s = carry
            xn = jax.lax.fori_loop(0, _PROJ_ITER, proj_body,
                                   x - _RELAX_LR * cost)
            return xn, s + xn

        _, s = jax.lax.fori_loop(
            0, _MAX_ITER, outer, (x0, jnp.zeros((o, p), dtype=jnp.float32)))
        ridx = s / jnp.float32(_MAX_ITER)

        logic = (ridx > 0.01).astype(jnp.float32)
        binx = ridx * logic
        binx_scr[...] = binx
        ms_ref[...] = jnp.max(jnp.clip(ridx, 0.0, 1.0) * sim, axis=1,
                              keepdims=True)
        ds_ref[...] = jnp.sum(ps_ref[...] * binx, axis=1, keepdims=True)

    @pl.when(ph == 1)
    def _phase1():
        x = binx_scr[...]
        xh = x.astype(jnp.bfloat16)
        xl = (x - xh.astype(jnp.float32)).astype(jnp.bfloat16)
        bbf = b8_scr[j].astype(jnp.bfloat16)
        dn = (((1,), (0,)), ((), ()))
        flat = (jax.lax.dot_general(xh, bbf, dn,
                                    preferred_element_type=jnp.float32)
                + jax.lax.dot_general(xl, bbf, dn,
                                      preferred_element_type=jnp.float32))
        out_ref[...] = flat.reshape(o, hb, w)


def kernel(proposed_feature, proposed_mask, template_feature,
           mask_last_occurence, proposal_score):
    p, d = proposed_feature.shape
    o = mask_last_occurence.shape[0]
    t = template_feature.shape[0]
    h, w = proposed_mask.shape[1], proposed_mask.shape[2]
    hw = h * w
    ch = _HB * w
    nc = h // _HB

    b2 = jax.lax.optimization_barrier(proposed_mask.reshape(p, hw))

    outmask, ms, ds = pl.pallas_call(
        _fused_body,
        grid=(2, nc),
        in_specs=[
            pl.BlockSpec((o, _HB, w), lambda ph, j: (0, j * (1 - ph), 0)),
            pl.BlockSpec((p, ch), lambda ph, j: (0, j * (1 - ph))),
            pl.BlockSpec((p, d), lambda ph, j: (0, 0)),
            pl.BlockSpec((t, o, d), lambda ph, j: (0, 0, 0)),
            pl.BlockSpec((1, p), lambda ph, j: (0, 0)),
        ],
        out_specs=[
            pl.BlockSpec((o, _HB, w), lambda ph, j: (0, j * ph, 0)),
            pl.BlockSpec((o, 1), lambda ph, j: (0, 0)),
            pl.BlockSpec((o, 1), lambda ph, j: (0, 0)),
        ],
        out_shape=[
            jax.ShapeDtypeStruct((o, h, w), jnp.float32),
            jax.ShapeDtypeStruct((o, 1), jnp.float32),
            jax.ShapeDtypeStruct((o, 1), jnp.float32),
        ],
        scratch_shapes=[
            pltpu.VMEM((nc, p, ch), jnp.int8),
            pltpu.VMEM((o + 8, p), jnp.float32),
            pltpu.VMEM((o, 1), jnp.float32),
            pltpu.VMEM((o, p), jnp.float32),
        ],
        compiler_params=pltpu.CompilerParams(
            dimension_semantics=("arbitrary", "arbitrary"),
            vmem_limit_bytes=_VMEM_LIMIT),
        name="match_model_fused",
    )(mask_last_occurence, b2, proposed_feature, template_feature,
      proposal_score.reshape(1, p))

    return (outmask, ms.reshape(o), ds.reshape(o))


# speedup vs baseline: 1.5388x; 1.0169x over previous
import jax
import jax.numpy as jnp
from jax.experimental import pallas as pl
from jax.experimental.pallas import tpu as pltpu

_SCORE_WEIGHT = 0.5
_MAX_ITER = 20
_PROJ_ITER = 5
_RELAX_LR = 0.1
_EPS = 1e-8

_HB = 16
_VMEM_LIMIT = 52 * 1024 * 1024


def _fused_body(a_ref, b_ref, pf_ref, tf_ref, ps_ref,
                out_ref, ms_ref, ds_ref,
                b8_scr, inter_scr, asum_scr, binx_scr):
    ph = pl.program_id(0)
    j = pl.program_id(1)
    o, hb, w = a_ref.shape
    p, ch = b_ref.shape
    half = p // 2

    @pl.when((ph == 0) & (j == 0))
    def _():
        inter_scr[...] = jnp.zeros_like(inter_scr)
        asum_scr[...] = jnp.zeros_like(asum_scr)

    @pl.when(ph == 0)
    def _phase0():
        a = a_ref[...].reshape(o, hb * w)
        lhs = jnp.concatenate(
            [a.astype(jnp.bfloat16), jnp.ones((8, ch), jnp.bfloat16)],
            axis=0)
        b8 = b_ref[...]
        bbf = b8.astype(jnp.bfloat16)
        acc = jax.lax.dot_general(lhs, bbf, (((1,), (1,)), ((), ())),
                                  preferred_element_type=jnp.float32)
        inter_scr[...] += acc
        asum_scr[...] += jnp.sum(a, axis=1, keepdims=True)
        b8_scr[j, :half, :] = b8[:half, :]
        b8_scr[j, half:, :] = b8[half:, :]

    @pl.when((ph == 1) & (j == 0))
    def _solve():
        inter = inter_scr[:o, :]
        bsum = inter_scr[o:o + 1, :]
        asum = asum_scr[...]
        union = asum + bsum - inter
        iou = inter / (union + _EPS)

        pf = pf_ref[...]
        kf = pf / (jnp.sqrt(jnp.sum(pf * pf, axis=1, keepdims=True)) + _EPS)
        tf = tf_ref[...]
        qn = jnp.sqrt(jnp.sum(tf * tf, axis=2, keepdims=True)) + _EPS
        qf = tf / qn
        qsum = jnp.sum(qf, axis=0)
        feature_sim = jax.lax.dot_general(
            qsum, kf, (((1,), (1,)), ((), ())),
            preferred_element_type=jnp.float32) / tf_ref.shape[0]

        sim = feature_sim * (1.0 - _SCORE_WEIGHT) + iou * _SCORE_WEIGHT
        cost = -sim

        x0 = jnp.full((o, p), 1.0 / p, dtype=jnp.float32)

        def proj_body(_, x):
            x = jnp.clip(x, 0.0, 1.0)
            return x / (jnp.sum(x, axis=1, keepdims=True) + _EPS)

        def outer(_, carry):
            x, s = carry
            xn = jax.lax.fori_loop(0, _PROJ_ITER, proj_body,
                                   x - _RELAX_LR * cost)
            return xn, s + xn

        _, s = jax.lax.fori_loop(
            0, _MAX_ITER, outer, (x0, jnp.zeros((o, p), dtype=jnp.float32)))
        ridx = s / jnp.float32(_MAX_ITER)

        logic = (ridx > 0.01).astype(jnp.float32)
        binx = ridx * logic
        binx_scr[...] = binx
        ms_ref[...] = jnp.max(jnp.clip(ridx, 0.0, 1.0) * sim, axis=1,
                              keepdims=True)
        ds_ref[...] = jnp.sum(ps_ref[...] * binx, axis=1, keepdims=True)

    @pl.when(ph == 1)
    def _phase1():
        x = binx_scr[...]
        xh = x.astype(jnp.bfloat16)
        xl = (x - xh.astype(jnp.float32)).astype(jnp.bfloat16)
        bbf = b8_scr[j].astype(jnp.bfloat16)
        dn = (((1,), (0,)), ((), ()))
        flat = (jax.lax.dot_general(xh, bbf, dn,
                                    preferred_element_type=jnp.float32)
                + jax.lax.dot_general(xl, bbf, dn,
                                      preferred_element_type=jnp.float32))
        out_ref[...] = flat.reshape(o, hb, w)


def kernel(proposed_feature, proposed_mask, template_feature,
           mask_last_occurence, proposal_score):
    p, d = proposed_feature.shape
    o = mask_last_occurence.shape[0]
    t = template_feature.shape[0]
    h, w = proposed_mask.shape[1], proposed_mask.shape[2]
    hw = h * w
    ch = _HB * w
    nc = h // _HB

    b2 = jax.lax.optimization_barrier(
        proposed_mask.reshape(p, hw).astype(jnp.int8))

    outmask, ms, ds = pl.pallas_call(
        _fused_body,
        grid=(2, nc),
        in_specs=[
            pl.BlockSpec((o, _HB, w), lambda ph, j: (0, j * (1 - ph), 0)),
            pl.BlockSpec((p, ch), lambda ph, j: (0, j * (1 - ph))),
            pl.BlockSpec((p, d), lambda ph, j: (0, 0)),
            pl.BlockSpec((t, o, d), lambda ph, j: (0, 0, 0)),
            pl.BlockSpec((1, p), lambda ph, j: (0, 0)),
        ],
        out_specs=[
            pl.BlockSpec((o, _HB, w), lambda ph, j: (0, j * ph, 0)),
            pl.BlockSpec((o, 1), lambda ph, j: (0, 0)),
            pl.BlockSpec((o, 1), lambda ph, j: (0, 0)),
        ],
        out_shape=[
            jax.ShapeDtypeStruct((o, h, w), jnp.float32),
            jax.ShapeDtypeStruct((o, 1), jnp.float32),
            jax.ShapeDtypeStruct((o, 1), jnp.float32),
        ],
        scratch_shapes=[
            pltpu.VMEM((nc, p, ch), jnp.int8),
            pltpu.VMEM((o + 8, p), jnp.float32),
            pltpu.VMEM((o, 1), jnp.float32),
            pltpu.VMEM((o, p), jnp.float32),
        ],
        compiler_params=pltpu.CompilerParams(
            dimension_semantics=("arbitrary", "arbitrary"),
            vmem_limit_bytes=_VMEM_LIMIT),
        name="match_model_fused",
    )(mask_last_occurence, b2, proposed_feature, template_feature,
      proposal_score.reshape(1, p))

    return (outmask, ms.reshape(o), ds.reshape(o))


# speedup vs baseline: 1.6195x; 1.0525x over previous
import jax
import jax.numpy as jnp
from jax.experimental import pallas as pl
from jax.experimental.pallas import tpu as pltpu

_SCORE_WEIGHT = 0.5
_MAX_ITER = 20
_PROJ_ITER = 5
_RELAX_LR = 0.1
_EPS = 1e-8

_HB = 16
_VMEM_LIMIT = 52 * 1024 * 1024


def _fused_body(a_ref, b_ref, pf_ref, tf_ref, ps_ref,
                out_ref, ms_ref, ds_ref,
                b8_scr, inter_scr, asum_scr, binx_scr):
    ph = pl.program_id(0)
    j = pl.program_id(1)
    o, hb, w = a_ref.shape
    p, ch = b_ref.shape
    half = p // 2

    @pl.when((ph == 0) & (j == 0))
    def _():
        inter_scr[...] = jnp.zeros_like(inter_scr)
        asum_scr[...] = jnp.zeros_like(asum_scr)

    @pl.when(ph == 0)
    def _phase0():
        a = a_ref[...].reshape(o, hb * w)
        lhs = jnp.concatenate(
            [a.astype(jnp.bfloat16), jnp.ones((8, ch), jnp.bfloat16)],
            axis=0)
        b8 = b_ref[...]
        bbf = b8.astype(jnp.bfloat16)
        acc = jax.lax.dot_general(lhs, bbf, (((1,), (1,)), ((), ())),
                                  preferred_element_type=jnp.float32)
        inter_scr[...] += acc
        asum_scr[...] += jnp.sum(a, axis=1, keepdims=True)
        b8_scr[j, :half, :] = b8[:half, :]
        b8_scr[j, half:, :] = b8[half:, :]

    @pl.when((ph == 1) & (j == 0))
    def _solve():
        inter = inter_scr[:o, :]
        bsum = inter_scr[o:o + 1, :]
        asum = asum_scr[...]
        union = asum + bsum - inter
        iou = inter / (union + _EPS)

        pf = pf_ref[...]
        kf = pf / (jnp.sqrt(jnp.sum(pf * pf, axis=1, keepdims=True)) + _EPS)
        tf = tf_ref[...]
        qn = jnp.sqrt(jnp.sum(tf * tf, axis=2, keepdims=True)) + _EPS
        qf = tf / qn
        qsum = jnp.sum(qf, axis=0)
        feature_sim = jax.lax.dot_general(
            qsum, kf, (((1,), (1,)), ((), ())),
            preferred_element_type=jnp.float32) / tf_ref.shape[0]

        sim = feature_sim * (1.0 - _SCORE_WEIGHT) + iou * _SCORE_WEIGHT
        cost = -sim

        x0 = jnp.full((o, p), 1.0 / p, dtype=jnp.float32)

        def proj_body(_, x):
            x = jnp.clip(x, 0.0, 1.0)
            return x / (jnp.sum(x, axis=1, keepdims=True) + _EPS)

        def outer(_, carry):
            x, s = carry
            xn = jax.lax.fori_loop(0, _PROJ_ITER, proj_body,
                                   x - _RELAX_LR * cost)
            return xn, s + xn

        _, s = jax.lax.fori_loop(
            0, _MAX_ITER, outer, (x0, jnp.zeros((o, p), dtype=jnp.float32)))
        ridx = s / jnp.float32(_MAX_ITER)

        logic = (ridx > 0.01).astype(jnp.float32)
        binx = ridx * logic
        binx_scr[...] = binx
        ms_ref[...] = jnp.max(jnp.clip(ridx, 0.0, 1.0) * sim, axis=1,
                              keepdims=True)
        ds_ref[...] = jnp.sum(ps_ref[...] * binx, axis=1, keepdims=True)

    @pl.when(ph == 1)
    def _phase1():
        x = binx_scr[...]
        xh = x.astype(jnp.bfloat16)
        xl = (x - xh.astype(jnp.float32)).astype(jnp.bfloat16)
        xs = jnp.concatenate([xh, xl], axis=0)
        bbf = b8_scr[j].astype(jnp.bfloat16)
        dn = (((1,), (0,)), ((), ()))
        both = jax.lax.dot_general(xs, bbf, dn,
                                   preferred_element_type=jnp.float32)
        flat = both[:o, :] + both[o:, :]
        out_ref[...] = flat.reshape(o, hb, w)


def kernel(proposed_feature, proposed_mask, template_feature,
           mask_last_occurence, proposal_score):
    p, d = proposed_feature.shape
    o = mask_last_occurence.shape[0]
    t = template_feature.shape[0]
    h, w = proposed_mask.shape[1], proposed_mask.shape[2]
    hw = h * w
    ch = _HB * w
    nc = h // _HB

    b2 = jax.lax.optimization_barrier(
        proposed_mask.reshape(p, hw).astype(jnp.int8))

    outmask, ms, ds = pl.pallas_call(
        _fused_body,
        grid=(2, nc),
        in_specs=[
            pl.BlockSpec((o, _HB, w), lambda ph, j: (0, j * (1 - ph), 0)),
            pl.BlockSpec((p, ch), lambda ph, j: (0, j * (1 - ph))),
            pl.BlockSpec((p, d), lambda ph, j: (0, 0)),
            pl.BlockSpec((t, o, d), lambda ph, j: (0, 0, 0)),
            pl.BlockSpec((1, p), lambda ph, j: (0, 0)),
        ],
        out_specs=[
            pl.BlockSpec((o, _HB, w), lambda ph, j: (0, j * ph, 0)),
            pl.BlockSpec((o, 1), lambda ph, j: (0, 0)),
            pl.BlockSpec((o, 1), lambda ph, j: (0, 0)),
        ],
        out_shape=[
            jax.ShapeDtypeStruct((o, h, w), jnp.float32),
            jax.ShapeDtypeStruct((o, 1), jnp.float32),
            jax.ShapeDtypeStruct((o, 1), jnp.float32),
        ],
        scratch_shapes=[
            pltpu.VMEM((nc, p, ch), jnp.int8),
            pltpu.VMEM((o + 8, p), jnp.float32),
            pltpu.VMEM((o, 1), jnp.float32),
            pltpu.VMEM((o, p), jnp.float32),
        ],
        compiler_params=pltpu.CompilerParams(
            dimension_semantics=("arbitrary", "arbitrary"),
            vmem_limit_bytes=_VMEM_LIMIT),
        name="match_model_fused",
    )(mask_last_occurence, b2, proposed_feature, template_feature,
      proposal_score.reshape(1, p))

    return (outmask, ms.reshape(o), ds.reshape(o))


# speedup vs baseline: 1.6550x; 1.0219x over previous
import jax
import jax.numpy as jnp
from jax.experimental import pallas as pl
from jax.experimental.pallas import tpu as pltpu

_SCORE_WEIGHT = 0.5
_MAX_ITER = 20
_PROJ_ITER = 5
_RELAX_LR = 0.1
_EPS = 1e-8

_HB = 24
_VMEM_LIMIT = 52 * 1024 * 1024


def _fused_body(a_ref, b_ref, pf_ref, tf_ref, ps_ref,
                out_ref, ms_ref, ds_ref,
                b8_scr, inter_scr, asum_scr, binx_scr):
    ph = pl.program_id(0)
    j = pl.program_id(1)
    o, hb, w = a_ref.shape
    p, ch = b_ref.shape
    half = p // 2

    @pl.when((ph == 0) & (j == 0))
    def _():
        inter_scr[...] = jnp.zeros_like(inter_scr)
        asum_scr[...] = jnp.zeros_like(asum_scr)

    @pl.when(ph == 0)
    def _phase0():
        a = a_ref[...].reshape(o, hb * w)
        lhs = jnp.concatenate(
            [a.astype(jnp.bfloat16), jnp.ones((8, ch), jnp.bfloat16)],
            axis=0)
        b8 = b_ref[...]
        bbf = b8.astype(jnp.bfloat16)
        acc = jax.lax.dot_general(lhs, bbf, (((1,), (1,)), ((), ())),
                                  preferred_element_type=jnp.float32)
        inter_scr[...] += acc
        asum_scr[...] += jnp.sum(a, axis=1, keepdims=True)
        b8_scr[j, :half, :] = b8[:half, :]
        b8_scr[j, half:, :] = b8[half:, :]

    @pl.when((ph == 1) & (j == 0))
    def _solve():
        inter = inter_scr[:o, :]
        bsum = inter_scr[o:o + 1, :]
        asum = asum_scr[...]
        union = asum + bsum - inter
        iou = inter / (union + _EPS)

        pf = pf_ref[...]
        kf = pf / (jnp.sqrt(jnp.sum(pf * pf, axis=1, keepdims=True)) + _EPS)
        tf = tf_ref[...]
        qn = jnp.sqrt(jnp.sum(tf * tf, axis=2, keepdims=True)) + _EPS
        qf = tf / qn
        qsum = jnp.sum(qf, axis=0)
        feature_sim = jax.lax.dot_general(
            qsum, kf, (((1,), (1,)), ((), ())),
            preferred_element_type=jnp.float32) / tf_ref.shape[0]

        sim = feature_sim * (1.0 - _SCORE_WEIGHT) + iou * _SCORE_WEIGHT
        cost = -sim

        x0 = jnp.full((o, p), 1.0 / p, dtype=jnp.float32)

        def proj_body(_, x):
            x = jnp.clip(x, 0.0, 1.0)
            return x / (jnp.sum(x, axis=1, keepdims=True) + _EPS)

        def outer(_, carry):
            x, s = carry
            xn = jax.lax.fori_loop(0, _PROJ_ITER, proj_body,
                                   x - _RELAX_LR * cost)
            return xn, s + xn

        _, s = jax.lax.fori_loop(
            0, _MAX_ITER, outer, (x0, jnp.zeros((o, p), dtype=jnp.float32)))
        ridx = s / jnp.float32(_MAX_ITER)

        logic = (ridx > 0.01).astype(jnp.float32)
        binx = ridx * logic
        binx_scr[...] = binx
        ms_ref[...] = jnp.max(jnp.clip(ridx, 0.0, 1.0) * sim, axis=1,
                              keepdims=True)
        ds_ref[...] = jnp.sum(ps_ref[...] * binx, axis=1, keepdims=True)

    @pl.when(ph == 1)
    def _phase1():
        x = binx_scr[...]
        xh = x.astype(jnp.bfloat16)
        xl = (x - xh.astype(jnp.float32)).astype(jnp.bfloat16)
        xs = jnp.concatenate([xh, xl], axis=0)
        bbf = b8_scr[j].astype(jnp.bfloat16)
        dn = (((1,), (0,)), ((), ()))
        both = jax.lax.dot_general(xs, bbf, dn,
                                   preferred_element_type=jnp.float32)
        flat = both[:o, :] + both[o:, :]
        out_ref[...] = flat.reshape(o, hb, w)


def kernel(proposed_feature, proposed_mask, template_feature,
           mask_last_occurence, proposal_score):
    p, d = proposed_feature.shape
    o = mask_last_occurence.shape[0]
    t = template_feature.shape[0]
    h, w = proposed_mask.shape[1], proposed_mask.shape[2]
    hw = h * w
    ch = _HB * w
    nc = h // _HB

    b2 = jax.lax.optimization_barrier(
        proposed_mask.reshape(p, hw).astype(jnp.int8))

    outmask, ms, ds = pl.pallas_call(
        _fused_body,
        grid=(2, nc),
        in_specs=[
            pl.BlockSpec((o, _HB, w), lambda ph, j: (0, j * (1 - ph), 0)),
            pl.BlockSpec((p, ch), lambda ph, j: (0, j * (1 - ph))),
            pl.BlockSpec((p, d), lambda ph, j: (0, 0)),
            pl.BlockSpec((t, o, d), lambda ph, j: (0, 0, 0)),
            pl.BlockSpec((1, p), lambda ph, j: (0, 0)),
        ],
        out_specs=[
            pl.BlockSpec((o, _HB, w), lambda ph, j: (0, j * ph, 0)),
            pl.BlockSpec((o, 1), lambda ph, j: (0, 0)),
            pl.BlockSpec((o, 1), lambda ph, j: (0, 0)),
        ],
        out_shape=[
            jax.ShapeDtypeStruct((o, h, w), jnp.float32),
            jax.ShapeDtypeStruct((o, 1), jnp.float32),
            jax.ShapeDtypeStruct((o, 1), jnp.float32),
        ],
        scratch_shapes=[
            pltpu.VMEM((nc, p, ch), jnp.int8),
            pltpu.VMEM((o + 8, p), jnp.float32),
            pltpu.VMEM((o, 1), jnp.float32),
            pltpu.VMEM((o, p), jnp.float32),
        ],
        compiler_params=pltpu.CompilerParams(
            dimension_semantics=("arbitrary", "arbitrary"),
            vmem_limit_bytes=_VMEM_LIMIT),
        name="match_model_fused",
    )(mask_last_occurence, b2, proposed_feature, template_feature,
      proposal_score.reshape(1, p))

    return (outmask, ms.reshape(o), ds.reshape(o))


# speedup vs baseline: 1.6809x; 1.0157x over previous
import jax
import jax.numpy as jnp
from jax.experimental import pallas as pl
from jax.experimental.pallas import tpu as pltpu

_SCORE_WEIGHT = 0.5
_MAX_ITER = 20
_PROJ_ITER = 5
_RELAX_LR = 0.1
_EPS = 1e-8

_HB = 40
_VMEM_LIMIT = 52 * 1024 * 1024


def _fused_body(a_ref, b_ref, pf_ref, tf_ref, ps_ref,
                out_ref, ms_ref, ds_ref,
                b8_scr, inter_scr, asum_scr, binx_scr):
    ph = pl.program_id(0)
    j = pl.program_id(1)
    o, hb, w = a_ref.shape
    p, ch = b_ref.shape
    half = p // 2

    @pl.when((ph == 0) & (j == 0))
    def _():
        inter_scr[...] = jnp.zeros_like(inter_scr)
        asum_scr[...] = jnp.zeros_like(asum_scr)

    @pl.when(ph == 0)
    def _phase0():
        a = a_ref[...].reshape(o, hb * w)
        lhs = jnp.concatenate(
            [a.astype(jnp.bfloat16), jnp.ones((8, ch), jnp.bfloat16)],
            axis=0)
        b8 = b_ref[...]
        bbf = b8.astype(jnp.bfloat16)
        acc = jax.lax.dot_general(lhs, bbf, (((1,), (1,)), ((), ())),
                                  preferred_element_type=jnp.float32)
        inter_scr[...] += acc
        asum_scr[...] += jnp.sum(a, axis=1, keepdims=True)
        b8_scr[j, :half, :] = b8[:half, :]
        b8_scr[j, half:, :] = b8[half:, :]

    @pl.when((ph == 1) & (j == 0))
    def _solve():
        inter = inter_scr[:o, :]
        bsum = inter_scr[o:o + 1, :]
        asum = asum_scr[...]
        union = asum + bsum - inter
        iou = inter / (union + _EPS)

        pf = pf_ref[...]
        kf = pf / (jnp.sqrt(jnp.sum(pf * pf, axis=1, keepdims=True)) + _EPS)
        tf = tf_ref[...]
        qn = jnp.sqrt(jnp.sum(tf * tf, axis=2, keepdims=True)) + _EPS
        qf = tf / qn
        qsum = jnp.sum(qf, axis=0)
        feature_sim = jax.lax.dot_general(
            qsum, kf, (((1,), (1,)), ((), ())),
            preferred_element_type=jnp.float32) / tf_ref.shape[0]

        sim = feature_sim * (1.0 - _SCORE_WEIGHT) + iou * _SCORE_WEIGHT
        cost = -sim

        x0 = jnp.full((o, p), 1.0 / p, dtype=jnp.float32)

        def proj_body(_, x):
            x = jnp.clip(x, 0.0, 1.0)
            return x / (jnp.sum(x, axis=1, keepdims=True) + _EPS)

        def outer(_, carry):
            x, s = carry
            xn = jax.lax.fori_loop(0, _PROJ_ITER, proj_body,
                                   x - _RELAX_LR * cost)
            return xn, s + xn

        _, s = jax.lax.fori_loop(
            0, _MAX_ITER, outer, (x0, jnp.zeros((o, p), dtype=jnp.float32)))
        ridx = s / jnp.float32(_MAX_ITER)

        logic = (ridx > 0.01).astype(jnp.float32)
        binx = ridx * logic
        binx_scr[...] = binx
        ms_ref[...] = jnp.max(jnp.clip(ridx, 0.0, 1.0) * sim, axis=1,
                              keepdims=True)
        ds_ref[...] = jnp.sum(ps_ref[...] * binx, axis=1, keepdims=True)

    @pl.when(ph == 1)
    def _phase1():
        x = binx_scr[...]
        xh = x.astype(jnp.bfloat16)
        xl = (x - xh.astype(jnp.float32)).astype(jnp.bfloat16)
        xs = jnp.concatenate([xh, xl], axis=0)
        bbf = b8_scr[j].astype(jnp.bfloat16)
        dn = (((1,), (0,)), ((), ()))
        both = jax.lax.dot_general(xs, bbf, dn,
                                   preferred_element_type=jnp.float32)
        flat = both[:o, :] + both[o:, :]
        out_ref[...] = flat.reshape(o, hb, w)


def kernel(proposed_feature, proposed_mask, template_feature,
           mask_last_occurence, proposal_score):
    p, d = proposed_feature.shape
    o = mask_last_occurence.shape[0]
    t = template_feature.shape[0]
    h, w = proposed_mask.shape[1], proposed_mask.shape[2]
    hw = h * w
    ch = _HB * w
    nc = h // _HB

    b2 = jax.lax.optimization_barrier(
        proposed_mask.reshape(p, hw).astype(jnp.int8))

    outmask, ms, ds = pl.pallas_call(
        _fused_body,
        grid=(2, nc),
        in_specs=[
            pl.BlockSpec((o, _HB, w), lambda ph, j: (0, j * (1 - ph), 0)),
            pl.BlockSpec((p, ch), lambda ph, j: (0, j * (1 - ph))),
            pl.BlockSpec((p, d), lambda ph, j: (0, 0)),
            pl.BlockSpec((t, o, d), lambda ph, j: (0, 0, 0)),
            pl.BlockSpec((1, p), lambda ph, j: (0, 0)),
        ],
        out_specs=[
            pl.BlockSpec((o, _HB, w), lambda ph, j: (0, j * ph, 0)),
            pl.BlockSpec((o, 1), lambda ph, j: (0, 0)),
            pl.BlockSpec((o, 1), lambda ph, j: (0, 0)),
        ],
        out_shape=[
            jax.ShapeDtypeStruct((o, h, w), jnp.float32),
            jax.ShapeDtypeStruct((o, 1), jnp.float32),
            jax.ShapeDtypeStruct((o, 1), jnp.float32),
        ],
        scratch_shapes=[
            pltpu.VMEM((nc, p, ch), jnp.int8),
            pltpu.VMEM((o + 8, p), jnp.float32),
            pltpu.VMEM((o, 1), jnp.float32),
            pltpu.VMEM((o, p), jnp.float32),
        ],
        compiler_params=pltpu.CompilerParams(
            dimension_semantics=("arbitrary", "arbitrary"),
            vmem_limit_bytes=_VMEM_LIMIT),
        name="match_model_fused",
    )(mask_last_occurence, b2, proposed_feature, template_feature,
      proposal_score.reshape(1, p))

    return (outmask, ms.reshape(o), ds.reshape(o))
